# baseline (speedup 1.0000x reference)
import functools
from functools import partial

import jax
import jax.numpy as jnp
from jax import lax
from jax.experimental import pallas as pl
from jax.experimental.pallas import tpu as pltpu
from jax.experimental.pallas import tpu_sc as plsc

f32 = jnp.float32
bf16 = jnp.bfloat16
i32 = jnp.int32

_N0 = 2048
_DIM = 512
_HID = 128
_HEADS = 2
_HD = 64
_K0R, _K0P = 1638, 1664
_K1R, _K1P = 982, 1024

_BM = 128


def _nb(n):
    return n // _BM


def _binarize_body(g_ref, a_ref, at_ref):
    a = (g_ref[...] != 0).astype(bf16)
    a_ref[...] = a
    at_ref[...] = a.T


def _binarize(g):
    n = g.shape[0]
    bs = 256 if n % 256 == 0 else n
    grid = (n // bs, n // bs)
    return pl.pallas_call(
        _binarize_body,
        grid=grid,
        in_specs=[pl.BlockSpec((bs, bs), lambda i, j: (i, j))],
        out_specs=[pl.BlockSpec((bs, bs), lambda i, j: (i, j)),
                   pl.BlockSpec((bs, bs), lambda i, j: (j, i))],
        out_shape=[jax.ShapeDtypeStruct((n, n), bf16),
                   jax.ShapeDtypeStruct((n, n), bf16)],
    )(g)


def _stats_body(nm1, a_ref, dinv_ref, s2_ref):
    s = jnp.sum(a_ref[...].astype(f32), axis=1, keepdims=True)
    dinv = lax.rsqrt(1.0 + s)
    s2 = jax.nn.sigmoid(3.0 * s / nm1)
    dinv_ref[...] = jnp.broadcast_to(dinv, dinv_ref.shape)
    s2_ref[...] = jnp.broadcast_to(s2, s2_ref.shape)


def _stats(a, n_real):
    n = a.shape[0]
    return pl.pallas_call(
        partial(_stats_body, float(n_real - 1)),
        in_specs=[pl.BlockSpec((n, n), lambda: (0, 0))],
        out_specs=[pl.BlockSpec((n, 128), lambda: (0, 0)),
                   pl.BlockSpec((n, 128), lambda: (0, 0))],
        out_shape=[jax.ShapeDtypeStruct((n, 128), f32),
                   jax.ShapeDtypeStruct((n, 128), f32)],
    )(a)


def _xw_body(n_real, has_s2, cast, *refs):
    if has_s2:
        x_ref, w_ref, s1_ref, s2_ref, o_ref = refs
    else:
        (x_ref, w_ref, s1_ref, o_ref), s2_ref = refs, None
    i = pl.program_id(0)
    x, w = x_ref[...], w_ref[...]
    if cast:
        x, w = x.astype(bf16), w.astype(bf16)
    z = jnp.dot(x, w, preferred_element_type=f32)
    scale = s1_ref[...][:, :1]
    if s2_ref is not None:
        scale = scale * s2_ref[...][:, :1]
    z = z * scale
    rows = i * _BM + lax.broadcasted_iota(i32, (_BM, 1), 0)
    o_ref[...] = jnp.where(rows < n_real, z, 0.0)


def _xw(x, w, scale1, scale2, n_real, cast=False):
    n = x.shape[0]
    d_in, d_out = w.shape
    specs = [pl.BlockSpec((_BM, d_in), lambda i: (i, 0)),
             pl.BlockSpec((d_in, d_out), lambda i: (0, 0)),
             pl.BlockSpec((_BM, 128), lambda i: (i, 0))]
    args = [x, w, scale1]
    body = partial(_xw_body, n_real, scale2 is not None, cast)
    if scale2 is not None:
        specs.append(pl.BlockSpec((_BM, 128), lambda i: (i, 0)))
        args.append(scale2)
    return pl.pallas_call(
        body, grid=(_nb(n),),
        in_specs=specs,
        out_specs=pl.BlockSpec((_BM, d_out), lambda i: (i, 0)),
        out_shape=jax.ShapeDtypeStruct((n, d_out), f32),
    )(*args)


def _adj_body(n_real, has_skip, has_org, cast, *refs):
    refs = list(refs)
    a_ref, z_ref, zd_ref, dinv_ref, b_ref = refs[:5]
    pos = 5
    skip_ref = refs[pos] if has_skip else None
    pos += int(has_skip)
    org_ref = refs[pos] if has_org else None
    pos += int(has_org)
    o_ref = refs[pos]
    o2_ref = refs[pos + 1] if has_org else None
    i = pl.program_id(0)
    if cast:
        acc = jnp.dot(a_ref[...].astype(bf16), z_ref[...].astype(bf16),
                      preferred_element_type=f32)
    else:
        acc = jnp.dot(a_ref[...].astype(f32), z_ref[...],
                      preferred_element_type=f32)
    acc = acc + zd_ref[...]
    out = jax.nn.relu(acc * dinv_ref[...][:, :1] + b_ref[...])
    if skip_ref is not None:
        out = out + skip_ref[...]
    rows = i * _BM + lax.broadcasted_iota(i32, (_BM, 1), 0)
    out = jnp.where(rows < n_real, out, 0.0)
    o_ref[...] = out
    if o2_ref is not None:
        o2_ref[...] = out + org_ref[...]


def _adj(a, z, dinv, b, n_real, skip=None, org=None, cast=False):
    n = a.shape[0]
    d = z.shape[1]
    specs = [pl.BlockSpec((_BM, n), lambda i: (i, 0)),
             pl.BlockSpec((n, d), lambda i: (0, 0)),
             pl.BlockSpec((_BM, d), lambda i: (i, 0)),
             pl.BlockSpec((_BM, 128), lambda i: (i, 0)),
             pl.BlockSpec((1, d), lambda i: (0, 0))]
    args = [a, z, z, dinv, b.reshape(1, d)]
    if skip is not None:
        specs.append(pl.BlockSpec((_BM, d), lambda i: (i, 0)))
        args.append(skip)
    out_specs = [pl.BlockSpec((_BM, d), lambda i: (i, 0))]
    out_shape = [jax.ShapeDtypeStruct((n, d), f32)]
    if org is not None:
        specs.append(pl.BlockSpec((_BM, d), lambda i: (i, 0)))
        args.append(org)
        out_specs.append(pl.BlockSpec((_BM, d), lambda i: (i, 0)))
        out_shape.append(jax.ShapeDtypeStruct((n, d), f32))
    body = partial(_adj_body, n_real, skip is not None, org is not None, cast)
    outs = pl.pallas_call(
        body, grid=(_nb(n),),
        in_specs=specs, out_specs=out_specs, out_shape=out_shape,
    )(*args)
    return outs if org is not None else outs[0]


def _qkv_body(n_real, x_ref, wq_ref, wk_ref, wv_ref, bq_ref, bk_ref, bv_ref,
              q_ref, k_ref, v_ref):
    i = pl.program_id(0)
    x = x_ref[...]
    rows = i * _BM + lax.broadcasted_iota(i32, (_BM, 1), 0)
    m = rows < n_real
    q = jnp.dot(x, wq_ref[...], preferred_element_type=f32) + bq_ref[...]
    k = jnp.dot(x, wk_ref[...], preferred_element_type=f32) + bk_ref[...]
    v = jnp.dot(x, wv_ref[...], preferred_element_type=f32) + bv_ref[...]
    q_ref[...] = jnp.where(m, q, 0.0)
    k_ref[...] = jnp.where(m, k, 0.0)
    v_ref[...] = jnp.where(m, v, 0.0)


def _qkv(x, p, n_real):
    n = x.shape[0]
    wspec = pl.BlockSpec((_DIM, _HID), lambda i: (0, 0))
    bspec = pl.BlockSpec((1, _HID), lambda i: (0, 0))
    ospec = pl.BlockSpec((_BM, _HID), lambda i: (i, 0))
    return pl.pallas_call(
        partial(_qkv_body, n_real), grid=(_nb(n),),
        in_specs=[pl.BlockSpec((_BM, _DIM), lambda i: (i, 0)),
                  wspec, wspec, wspec, bspec, bspec, bspec],
        out_specs=[ospec, ospec, ospec],
        out_shape=[jax.ShapeDtypeStruct((n, _HID), f32)] * 3,
    )(x, p["Wq"], p["Wk"], p["Wv"], p["bq"].reshape(1, _HID),
      p["bk"].reshape(1, _HID), p["bv"].reshape(1, _HID))


def _attn_body(n_real, q_ref, k_ref, v_ref, o_ref):
    cols = lax.broadcasted_iota(i32, (1, k_ref.shape[0]), 1)
    for hh in range(_HEADS):
        sl = slice(hh * _HD, (hh + 1) * _HD)
        qh = q_ref[:, sl]
        kh = k_ref[:, sl]
        vh = v_ref[:, sl]
        s = lax.dot_general(qh, kh, (((1,), (1,)), ((), ())),
                            preferred_element_type=f32) * (1.0 / 8.0)
        s = jnp.where(cols < n_real, s, -1e30)
        m = jnp.max(s, axis=1, keepdims=True)
        p = jnp.exp(s - m)
        l = jnp.sum(p, axis=1, keepdims=True)
        o_ref[:, sl] = jnp.dot(p, vh, preferred_element_type=f32) / l


def _attn(q, k, v, n_real):
    n = q.shape[0]
    full = pl.BlockSpec((n, _HID), lambda i: (0, 0))
    return pl.pallas_call(
        partial(_attn_body, n_real), grid=(_nb(n),),
        in_specs=[pl.BlockSpec((_BM, _HID), lambda i: (i, 0)), full, full],
        out_specs=pl.BlockSpec((_BM, _HID), lambda i: (i, 0)),
        out_shape=jax.ShapeDtypeStruct((n, _HID), f32),
    )(q, k, v)


def _combine_body(n_real, ctx_ref, wd_ref, s2_ref, bd_ref, va_ref, vb_ref,
                  sc_ref):
    n = ctx_ref.shape[0]
    rows = lax.broadcasted_iota(i32, (n, 1), 0)
    valid = rows < n_real
    raw = jnp.sum(ctx_ref[...] * wd_ref[...], axis=1, keepdims=True) \
        + bd_ref[0, 0]
    s1 = jnp.where(valid, jax.nn.sigmoid(raw), 0.0)
    s2 = jnp.where(valid, s2_ref[...][:, :1], 0.0)
    sn1 = s1 / jnp.max(s1)
    sn2 = s2 / jnp.max(s2)
    a0 = jax.nn.sigmoid(sn1 * va_ref[0, 0] + sn2 * va_ref[1, 0] + vb_ref[0, 0])
    a1 = jax.nn.sigmoid(sn1 * va_ref[0, 1] + sn2 * va_ref[1, 1] + vb_ref[0, 1])
    mx = jnp.maximum(a0, a1)
    e0 = jnp.exp(a0 - mx)
    e1 = jnp.exp(a1 - mx)
    sc = jax.nn.sigmoid((sn1 * e0 + sn2 * e1) / (e0 + e1))
    sc = jnp.where(valid, sc, -1e30)
    sc_ref[...] = jnp.broadcast_to(sc, sc_ref.shape)


def _combine(ctx, s2_col, p, n_real):
    n = ctx.shape[0]
    return pl.pallas_call(
        partial(_combine_body, n_real),
        in_specs=[pl.BlockSpec((n, _HID), lambda: (0, 0)),
                  pl.BlockSpec((1, _HID), lambda: (0, 0)),
                  pl.BlockSpec((n, 128), lambda: (0, 0)),
                  pl.BlockSpec(memory_space=pltpu.SMEM),
                  pl.BlockSpec(memory_space=pltpu.SMEM),
                  pl.BlockSpec(memory_space=pltpu.SMEM)],
        out_specs=pl.BlockSpec((n, 128), lambda: (0, 0)),
        out_shape=jax.ShapeDtypeStruct((n, 128), f32),
    )(ctx, p["Wd"].reshape(1, _HID), s2_col, p["bd"].reshape(1, 1),
      p["view_att"], p["view_bias"].reshape(1, 2))


def _rank_body(kk_real, sc_col_ref, sc_row_ref, r_ref, pos_ref):
    i = pl.program_id(0)
    s_i = sc_col_ref[...][:, :1]
    s_j = sc_row_ref[...]
    jj = lax.broadcasted_iota(i32, s_j.shape, 1)
    ii = i * _BM + lax.broadcasted_iota(i32, (_BM, 1), 0)
    beats = (s_j > s_i) | ((s_j == s_i) & (jj < ii))
    r = jnp.sum(beats.astype(i32), axis=1, keepdims=True)
    r_ref[...] = jnp.broadcast_to(r, r_ref.shape)
    pos = jnp.where(r < kk_real, r, kk_real)
    pos_ref[...] = jnp.broadcast_to(pos, pos_ref.shape)


def _rank(sc_col, sc_row, kk_real):
    n = sc_col.shape[0]
    return pl.pallas_call(
        partial(_rank_body, kk_real), grid=(_nb(n),),
        in_specs=[pl.BlockSpec((_BM, 128), lambda i: (i, 0)),
                  pl.BlockSpec((1, n), lambda i: (0, 0))],
        out_specs=[pl.BlockSpec((_BM, 128), lambda i: (i, 0)),
                   pl.BlockSpec((_BM, 128), lambda i: (i, 0))],
        out_shape=[jax.ShapeDtypeStruct((n, 128), i32),
                   jax.ShapeDtypeStruct((n, 128), i32)],
    )(sc_col, sc_row)


def _idxsel_body(rank_row_ref, idx_ref):
    i = pl.program_id(0)
    rr = rank_row_ref[...]
    r_col = i * _BM + lax.broadcasted_iota(i32, (_BM, 1), 0)
    ii = lax.broadcasted_iota(i32, rr.shape, 1)
    hit = jnp.where(rr == r_col, ii, 0)
    idx = jnp.sum(hit, axis=1, keepdims=True)
    idx_ref[...] = jnp.broadcast_to(idx, idx_ref.shape)


def _idxsel(rank_row, kk_pad):
    n = rank_row.shape[1]
    return pl.pallas_call(
        _idxsel_body, grid=(kk_pad // _BM,),
        in_specs=[pl.BlockSpec((1, n), lambda i: (0, 0))],
        out_specs=pl.BlockSpec((_BM, 128), lambda i: (i, 0)),
        out_shape=jax.ShapeDtypeStruct((kk_pad, 128), i32),
    )(rank_row)


def _a2_body(kk_real, out_dtype, a_ref, b_ref, o_ref, ot_ref):
    i = pl.program_id(0)
    j = pl.program_id(1)
    acc = lax.dot_general(a_ref[...].astype(bf16), b_ref[...].astype(bf16),
                          (((1,), (1,)), ((), ())),
                          preferred_element_type=f32)
    rows = i * _BM + lax.broadcasted_iota(i32, (_BM, 1), 0)
    cols = j * _BM + lax.broadcasted_iota(i32, (1, _BM), 1)
    bin_ = ((acc > 0.5) & (rows < kk_real) & (cols < kk_real)).astype(out_dtype)
    o_ref[...] = bin_
    ot_ref[...] = bin_.T


def _a2(ag, atg, kk_real, out_dtype=bf16):
    kk, w = ag.shape
    return pl.pallas_call(
        partial(_a2_body, kk_real, out_dtype), grid=(_nb(kk), _nb(kk)),
        in_specs=[pl.BlockSpec((_BM, w), lambda i, j: (i, 0)),
                  pl.BlockSpec((_BM, w), lambda i, j: (j, 0))],
        out_specs=[pl.BlockSpec((_BM, _BM), lambda i, j: (i, j)),
                   pl.BlockSpec((_BM, _BM), lambda i, j: (j, i))],
        out_shape=[jax.ShapeDtypeStruct((kk, kk), out_dtype),
                   jax.ShapeDtypeStruct((kk, kk), out_dtype)],
    )(ag, atg)


def _sc_mesh():
    return plsc.VectorSubcoreMesh(core_axis_name="c", subcore_axis_name="s",
                                  num_cores=2, num_subcores=16)


def _gather_sc(table, idx, rows_per_tile=64):
    if table.dtype == bf16:
        n_r, n_c = table.shape
        t32 = lax.bitcast_convert_type(table.reshape(n_r, n_c // 2, 2),
                                       jnp.uint32)
        out32 = _gather_sc(t32, idx, rows_per_tile)
        return lax.bitcast_convert_type(out32, bf16).reshape(
            idx.shape[0], n_c)
    out_rows = idx.shape[0]
    n_tiles = out_rows // rows_per_tile
    row_w = table.shape[1]
    dtype = table.dtype
    idx2d = idx.reshape(n_tiles, rows_per_tile)

    @partial(pl.kernel,
             out_type=jax.ShapeDtypeStruct((out_rows, row_w), dtype),
             mesh=_sc_mesh(),
             scratch_types=[pltpu.VMEM((rows_per_tile,), i32),
                            pltpu.VMEM((rows_per_tile, row_w), dtype),
                            pltpu.SemaphoreType.DMA])
    def k(tab_hbm, idx_hbm, out_hbm, idx_v, rows_v, sem):
        wid = lax.axis_index("s") * 2 + lax.axis_index("c")

        @pl.when(wid < n_tiles)
        def _():
            pltpu.sync_copy(idx_hbm.at[wid], idx_v)
            pltpu.async_copy(tab_hbm.at[idx_v], rows_v, sem).wait()
            pltpu.sync_copy(
                rows_v, out_hbm.at[pl.ds(wid * rows_per_tile, rows_per_tile)])

    return k(table, idx2d)


def _gcn(a, x, dinv, w, b, n_real, scale2=None, skip=None, org=None,
         cast=False):
    z = _xw(x, w, dinv, scale2, n_real, cast=cast)
    return _adj(a, z, dinv, b, n_real, skip=skip, org=org, cast=cast)


def _pool_scores(hh, p, s2_col, n_real):
    q, k, v = _qkv(hh, p, n_real)
    ctx = _attn(q, k, v, n_real)
    return _combine(ctx, s2_col, p, n_real)


def kernel(g, h, params):
    g = jnp.asarray(g, f32)
    h = jnp.asarray(h, f32)

    a0, at0 = _binarize(g)
    dinv0, s2c0 = _stats(a0, _N0)
    p0 = params["down0"]
    h1 = _gcn(a0, h, dinv0, p0["W"], p0["b"], _N0)
    sc0 = _pool_scores(h1, params["pool0"], s2c0, _N0)
    r0, posc0 = _rank(sc0, sc0[:, 0][None, :], _K0R)
    idx0 = _idxsel(r0[:, 0][None, :], _K0P)[:, 0]
    pos0 = posc0[:, 0]

    vals0 = _gather_sc(sc0, idx0)
    nh1 = _gather_sc(h1, idx0)
    ag0 = _gather_sc(a0, idx0)
    atg0 = _gather_sc(at0, idx0)
    a1, at1 = _a2(ag0, atg0, _K0R, out_dtype=f32)

    dinv1, s2c1 = _stats(a1, _K0R)
    p1 = params["down1"]
    h2 = _gcn(a1, nh1, dinv1, p1["W"], p1["b"], _K0R, scale2=vals0)
    sc1 = _pool_scores(h2, params["pool1"], s2c1, _K0R)
    r1, posc1 = _rank(sc1, sc1[:, 0][None, :], _K1R)
    idx1 = _idxsel(r1[:, 0][None, :], _K1P)[:, 0]
    pos1 = posc1[:, 0]

    vals1 = _gather_sc(sc1, idx1)
    nh2 = _gather_sc(h2, idx1)
    ag1 = _gather_sc(a1, idx1)
    atg1 = _gather_sc(at1, idx1)
    a2_, _at2 = _a2(ag1, atg1, _K1R)

    dinv2, _s2u = _stats(a2_, _K1R)
    pb = params["bottom"]
    hb = _gcn(a2_, nh2, dinv2, pb["W"], pb["b"], _K1R, scale2=vals1, cast=True)

    u1 = _gather_sc(hb, pos1)
    pu0 = params["up0"]
    hs0 = _gcn(a1, u1, dinv1, pu0["W"], pu0["b"], _K0R, skip=h2, cast=True)

    u0 = _gather_sc(hs0, pos0)
    pu1 = params["up1"]
    hs1, hs2 = _gcn(a0, u0, dinv0, pu1["W"], pu1["b"], _N0, skip=h1, org=h,
                    cast=True)

    return (hs0[:_K0R], hs1, hs2)


# speedup vs baseline: 2.2032x; 2.2032x over previous
from functools import partial

import jax
import jax.numpy as jnp
from jax import lax
from jax.experimental import pallas as pl
from jax.experimental.pallas import tpu as pltpu
from jax.experimental.pallas import tpu_sc as plsc

f32 = jnp.float32
bf16 = jnp.bfloat16
i32 = jnp.int32

_N = 2048
_DIM = 512
_HID = 128
_HEADS = 2
_HD = 64
_K0R, _K0P = 1638, 1664
_K1R = 982

_BM = 128
_NB = _N // _BM


def _binarize_body(g_ref, a_ref):
    a_ref[...] = (g_ref[...] != 0).astype(bf16)


def _binarize(g):
    bs = 256 if _N % 256 == 0 else _N
    return pl.pallas_call(
        _binarize_body,
        grid=(_N // bs, _N // bs),
        in_specs=[pl.BlockSpec((bs, bs), lambda i, j: (i, j))],
        out_specs=pl.BlockSpec((bs, bs), lambda i, j: (i, j)),
        out_shape=jax.ShapeDtypeStruct((_N, _N), bf16),
    )(g)


def _stats_body(nm1, has_m, *refs):
    if has_m:
        a_ref, m_ref, dinv_ref, s2_ref = refs
    else:
        (a_ref, dinv_ref, s2_ref), m_ref = refs, None
    s = jnp.sum(a_ref[...].astype(f32), axis=1, keepdims=True)
    dinv = lax.rsqrt(1.0 + s)
    if m_ref is not None:
        dinv = dinv * m_ref[...][:, :1]
    s2 = jax.nn.sigmoid(3.0 * s / nm1)
    dinv_ref[...] = jnp.broadcast_to(dinv, dinv_ref.shape)
    s2_ref[...] = jnp.broadcast_to(s2, s2_ref.shape)


def _stats(a, n_real, m_col=None):
    specs = [pl.BlockSpec((_N, _N), lambda: (0, 0))]
    args = [a]
    if m_col is not None:
        specs.append(pl.BlockSpec((_N, 128), lambda: (0, 0)))
        args.append(m_col)
    return pl.pallas_call(
        partial(_stats_body, float(n_real - 1), m_col is not None),
        in_specs=specs,
        out_specs=[pl.BlockSpec((_N, 128), lambda: (0, 0)),
                   pl.BlockSpec((_N, 128), lambda: (0, 0))],
        out_shape=[jax.ShapeDtypeStruct((_N, 128), f32),
                   jax.ShapeDtypeStruct((_N, 128), f32)],
    )(*args)


def _xw_body(has_s2, cast, *refs):
    if has_s2:
        x_ref, w_ref, s1_ref, s2_ref, o_ref = refs
    else:
        (x_ref, w_ref, s1_ref, o_ref), s2_ref = refs, None
    x, w = x_ref[...], w_ref[...]
    if cast:
        x, w = x.astype(bf16), w.astype(bf16)
    z = jnp.dot(x, w, preferred_element_type=f32)
    scale = s1_ref[...][:, :1]
    if s2_ref is not None:
        scale = scale * s2_ref[...][:, :1]
    o_ref[...] = z * scale


def _xw(x, w, scale1, scale2=None, cast=False):
    d_in, d_out = w.shape
    specs = [pl.BlockSpec((_BM, d_in), lambda i: (i, 0)),
             pl.BlockSpec((d_in, d_out), lambda i: (0, 0)),
             pl.BlockSpec((_BM, 128), lambda i: (i, 0))]
    args = [x, w, scale1]
    if scale2 is not None:
        specs.append(pl.BlockSpec((_BM, 128), lambda i: (i, 0)))
        args.append(scale2)
    return pl.pallas_call(
        partial(_xw_body, scale2 is not None, cast), grid=(_NB,),
        in_specs=specs,
        out_specs=pl.BlockSpec((_BM, d_out), lambda i: (i, 0)),
        out_shape=jax.ShapeDtypeStruct((_N, d_out), f32),
    )(*args)


def _adj_body(has_m, has_skip, has_org, cast, *refs):
    refs = list(refs)
    a_ref, z_ref, zd_ref, dinv_ref, b_ref = refs[:5]
    pos = 5
    m_ref = refs[pos] if has_m else None
    pos += int(has_m)
    skip_ref = refs[pos] if has_skip else None
    pos += int(has_skip)
    org_ref = refs[pos] if has_org else None
    pos += int(has_org)
    o_ref = refs[pos]
    o2_ref = refs[pos + 1] if has_org else None
    if cast:
        acc = jnp.dot(a_ref[...].astype(bf16), z_ref[...].astype(bf16),
                      preferred_element_type=f32)
    else:
        acc = jnp.dot(a_ref[...].astype(f32), z_ref[...],
                      preferred_element_type=f32)
    acc = acc + zd_ref[...]
    out = jax.nn.relu(acc * dinv_ref[...][:, :1] + b_ref[...])
    if m_ref is not None:
        out = out * m_ref[...][:, :1]
    if skip_ref is not None:
        out = out + skip_ref[...]
    o_ref[...] = out
    if o2_ref is not None:
        o2_ref[...] = out + org_ref[...]


def _adj(a, z, dinv, b, m_col=None, skip=None, org=None, cast=False):
    d = z.shape[1]
    specs = [pl.BlockSpec((_BM, _N), lambda i: (i, 0)),
             pl.BlockSpec((_N, d), lambda i: (0, 0)),
             pl.BlockSpec((_BM, d), lambda i: (i, 0)),
             pl.BlockSpec((_BM, 128), lambda i: (i, 0)),
             pl.BlockSpec((1, d), lambda i: (0, 0))]
    args = [a, z, z, dinv, b.reshape(1, d)]
    for extra in (m_col, skip, org):
        if extra is not None:
            specs.append(pl.BlockSpec((_BM, extra.shape[1]), lambda i: (i, 0)))
            args.append(extra)
    out_specs = [pl.BlockSpec((_BM, d), lambda i: (i, 0))]
    out_shape = [jax.ShapeDtypeStruct((_N, d), f32)]
    if org is not None:
        out_specs.append(pl.BlockSpec((_BM, d), lambda i: (i, 0)))
        out_shape.append(jax.ShapeDtypeStruct((_N, d), f32))
    body = partial(_adj_body, m_col is not None, skip is not None,
                   org is not None, cast)
    outs = pl.pallas_call(
        body, grid=(_NB,),
        in_specs=specs, out_specs=out_specs, out_shape=out_shape,
    )(*args)
    return outs if org is not None else outs[0]


def _qkv_body(x_ref, wq_ref, wk_ref, wv_ref, bq_ref, bk_ref, bv_ref,
              q_ref, k_ref, v_ref):
    x = x_ref[...]
    q_ref[...] = jnp.dot(x, wq_ref[...], preferred_element_type=f32) \
        + bq_ref[...]
    k_ref[...] = jnp.dot(x, wk_ref[...], preferred_element_type=f32) \
        + bk_ref[...]
    v_ref[...] = jnp.dot(x, wv_ref[...], preferred_element_type=f32) \
        + bv_ref[...]


def _qkv(x, p):
    wspec = pl.BlockSpec((_DIM, _HID), lambda i: (0, 0))
    bspec = pl.BlockSpec((1, _HID), lambda i: (0, 0))
    ospec = pl.BlockSpec((_BM, _HID), lambda i: (i, 0))
    return pl.pallas_call(
        _qkv_body, grid=(_NB,),
        in_specs=[pl.BlockSpec((_BM, _DIM), lambda i: (i, 0)),
                  wspec, wspec, wspec, bspec, bspec, bspec],
        out_specs=[ospec, ospec, ospec],
        out_shape=[jax.ShapeDtypeStruct((_N, _HID), f32)] * 3,
    )(x, p["Wq"], p["Wk"], p["Wv"], p["bq"].reshape(1, _HID),
      p["bk"].reshape(1, _HID), p["bv"].reshape(1, _HID))


def _attn_body(has_m, *refs):
    if has_m:
        q_ref, k_ref, v_ref, m_ref, o_ref = refs
    else:
        (q_ref, k_ref, v_ref, o_ref), m_ref = refs, None
    for hh in range(_HEADS):
        sl = slice(hh * _HD, (hh + 1) * _HD)
        s = lax.dot_general(q_ref[:, sl], k_ref[:, sl],
                            (((1,), (1,)), ((), ())),
                            preferred_element_type=f32) * (1.0 / 8.0)
        if m_ref is not None:
            s = jnp.where(m_ref[...] > 0, s, -1e30)
        mx = jnp.max(s, axis=1, keepdims=True)
        p = jnp.exp(s - mx)
        l = jnp.sum(p, axis=1, keepdims=True)
        o_ref[:, sl] = jnp.dot(p, v_ref[:, sl],
                               preferred_element_type=f32) / l


def _attn(q, k, v, m_row=None):
    full = pl.BlockSpec((_N, _HID), lambda i: (0, 0))
    specs = [pl.BlockSpec((_BM, _HID), lambda i: (i, 0)), full, full]
    args = [q, k, v]
    if m_row is not None:
        specs.append(pl.BlockSpec((1, _N), lambda i: (0, 0)))
        args.append(m_row)
    return pl.pallas_call(
        partial(_attn_body, m_row is not None), grid=(_NB,),
        in_specs=specs,
        out_specs=pl.BlockSpec((_BM, _HID), lambda i: (i, 0)),
        out_shape=jax.ShapeDtypeStruct((_N, _HID), f32),
    )(*args)


def _combine_body(has_m, *refs):
    if has_m:
        ctx_ref, wd_ref, s2_ref, m_ref, bd_ref, va_ref, vb_ref, sc_ref = refs
    else:
        (ctx_ref, wd_ref, s2_ref, bd_ref, va_ref, vb_ref, sc_ref), m_ref = \
            refs, None
    raw = jnp.sum(ctx_ref[...] * wd_ref[...], axis=1, keepdims=True) \
        + bd_ref[0, 0]
    s1 = jax.nn.sigmoid(raw)
    s2 = s2_ref[...][:, :1]
    if m_ref is not None:
        valid = m_ref[...][:, :1] > 0
        s1 = jnp.where(valid, s1, 0.0)
        s2 = jnp.where(valid, s2, 0.0)
    sn1 = s1 / jnp.max(s1)
    sn2 = s2 / jnp.max(s2)
    a0 = jax.nn.sigmoid(sn1 * va_ref[0, 0] + sn2 * va_ref[1, 0] + vb_ref[0, 0])
    a1 = jax.nn.sigmoid(sn1 * va_ref[0, 1] + sn2 * va_ref[1, 1] + vb_ref[0, 1])
    mx = jnp.maximum(a0, a1)
    e0 = jnp.exp(a0 - mx)
    e1 = jnp.exp(a1 - mx)
    sc = jax.nn.sigmoid((sn1 * e0 + sn2 * e1) / (e0 + e1))
    if m_ref is not None:
        sc = jnp.where(valid, sc, -1e30)
    sc_ref[...] = jnp.broadcast_to(sc, sc_ref.shape)


def _combine(ctx, s2_col, p, m_col=None):
    specs = [pl.BlockSpec((_N, _HID), lambda: (0, 0)),
             pl.BlockSpec((1, _HID), lambda: (0, 0)),
             pl.BlockSpec((_N, 128), lambda: (0, 0))]
    args = [ctx, p["Wd"].reshape(1, _HID), s2_col]
    if m_col is not None:
        specs.append(pl.BlockSpec((_N, 128), lambda: (0, 0)))
        args.append(m_col)
    specs += [pl.BlockSpec(memory_space=pltpu.SMEM)] * 3
    args += [p["bd"].reshape(1, 1), p["view_att"], p["view_bias"].reshape(1, 2)]
    return pl.pallas_call(
        partial(_combine_body, m_col is not None),
        in_specs=specs,
        out_specs=pl.BlockSpec((_N, 128), lambda: (0, 0)),
        out_shape=jax.ShapeDtypeStruct((_N, 128), f32),
    )(*args)


def _rank_body(kk_real, sc_col_ref, sc_row_ref, r_ref, m_ref):
    i = pl.program_id(0)
    s_i = sc_col_ref[...][:, :1]
    s_j = sc_row_ref[...]
    jj = lax.broadcasted_iota(i32, s_j.shape, 1)
    ii = i * _BM + lax.broadcasted_iota(i32, (_BM, 1), 0)
    beats = (s_j > s_i) | ((s_j == s_i) & (jj < ii))
    r = jnp.sum(beats.astype(i32), axis=1, keepdims=True)
    r_ref[...] = jnp.broadcast_to(r, r_ref.shape)
    m = (r < kk_real).astype(f32)
    m_ref[...] = jnp.broadcast_to(m, m_ref.shape)


def _rank(sc_col, sc_row, kk_real):
    return pl.pallas_call(
        partial(_rank_body, kk_real), grid=(_NB,),
        in_specs=[pl.BlockSpec((_BM, 128), lambda i: (i, 0)),
                  pl.BlockSpec((1, _N), lambda i: (0, 0))],
        out_specs=[pl.BlockSpec((_BM, 128), lambda i: (i, 0)),
                   pl.BlockSpec((_BM, 128), lambda i: (i, 0))],
        out_shape=[jax.ShapeDtypeStruct((_N, 128), i32),
                   jax.ShapeDtypeStruct((_N, 128), f32)],
    )(sc_col, sc_row)


def _idxsel_body(rank_row_ref, idx_ref):
    i = pl.program_id(0)
    rr = rank_row_ref[...]
    r_col = i * _BM + lax.broadcasted_iota(i32, (_BM, 1), 0)
    ii = lax.broadcasted_iota(i32, rr.shape, 1)
    hit = jnp.where(rr == r_col, ii, 0)
    idx = jnp.sum(hit, axis=1, keepdims=True)
    idx_ref[...] = jnp.broadcast_to(idx, idx_ref.shape)


def _idxsel(rank_row, kk_pad):
    return pl.pallas_call(
        _idxsel_body, grid=(kk_pad // _BM,),
        in_specs=[pl.BlockSpec((1, _N), lambda i: (0, 0))],
        out_specs=pl.BlockSpec((_BM, 128), lambda i: (i, 0)),
        out_shape=jax.ShapeDtypeStruct((kk_pad, 128), i32),
    )(rank_row)


def _a2_body(a_ref, ac_ref, m_ref, mr_ref, o_ref):
    acc = jnp.dot(a_ref[...], ac_ref[...], preferred_element_type=f32)
    keep = (acc > 0.5) & (m_ref[...][:, :1] > 0) & (mr_ref[...] > 0)
    o_ref[...] = keep.astype(bf16)


def _a2(a, m_col, m_row):
    return pl.pallas_call(
        _a2_body, grid=(_NB,),
        in_specs=[pl.BlockSpec((_BM, _N), lambda i: (i, 0)),
                  pl.BlockSpec((_N, _N), lambda i: (0, 0)),
                  pl.BlockSpec((_BM, 128), lambda i: (i, 0)),
                  pl.BlockSpec((1, _N), lambda i: (0, 0))],
        out_specs=pl.BlockSpec((_BM, _N), lambda i: (i, 0)),
        out_shape=jax.ShapeDtypeStruct((_N, _N), bf16),
    )(a, a, m_col, m_row)


def _sc_mesh():
    return plsc.VectorSubcoreMesh(core_axis_name="c", subcore_axis_name="s",
                                  num_cores=2, num_subcores=16)


def _gather_sc(table, idx, rows_per_tile=64):
    out_rows = idx.shape[0]
    n_tiles = out_rows // rows_per_tile
    row_w = table.shape[1]
    dtype = table.dtype
    idx2d = idx.reshape(n_tiles, rows_per_tile)

    @partial(pl.kernel,
             out_type=jax.ShapeDtypeStruct((out_rows, row_w), dtype),
             mesh=_sc_mesh(),
             scratch_types=[pltpu.VMEM((rows_per_tile,), i32),
                            pltpu.VMEM((rows_per_tile, row_w), dtype),
                            pltpu.SemaphoreType.DMA])
    def k(tab_hbm, idx_hbm, out_hbm, idx_v, rows_v, sem):
        wid = lax.axis_index("s") * 2 + lax.axis_index("c")

        @pl.when(wid < n_tiles)
        def _():
            pltpu.sync_copy(idx_hbm.at[wid], idx_v)
            pltpu.async_copy(tab_hbm.at[idx_v], rows_v, sem).wait()
            pltpu.sync_copy(
                rows_v, out_hbm.at[pl.ds(wid * rows_per_tile, rows_per_tile)])

    return k(table, idx2d)


def _gcn(a, x, dinv, w, b, scale2=None, m_col=None, skip=None, org=None,
         cast=False):
    z = _xw(x, w, dinv, scale2, cast=cast)
    return _adj(a, z, dinv, b, m_col=m_col, skip=skip, org=org, cast=cast)


def _pool_scores(hh, p, s2_col, m_col=None, m_row=None):
    q, k, v = _qkv(hh, p)
    ctx = _attn(q, k, v, m_row)
    return _combine(ctx, s2_col, p, m_col)


def _row(col):
    return col[:, 0][None, :]


def kernel(g, h, params):
    g = jnp.asarray(g, f32)
    h = jnp.asarray(h, f32)

    a0 = _binarize(g)
    dinv0, s2c0 = _stats(a0, _N)
    p0 = params["down0"]
    h1 = _gcn(a0, h, dinv0, p0["W"], p0["b"])
    sc0 = _pool_scores(h1, params["pool0"], s2c0)
    r0, m1 = _rank(sc0, _row(sc0), _K0R)
    idx0 = _idxsel(_row(r0), _K0P)[:, 0]
    m1row = _row(m1)

    a1 = _a2(a0, m1, m1row)

    dinv1, s2c1 = _stats(a1, _K0R, m1)
    p1 = params["down1"]
    h2 = _gcn(a1, h1, dinv1, p1["W"], p1["b"], scale2=sc0, m_col=m1)
    sc1 = _pool_scores(h2, params["pool1"], s2c1, m_col=m1, m_row=m1row)
    _r1, m2 = _rank(sc1, _row(sc1), _K1R)

    a2_ = _a2(a1, m2, _row(m2))

    dinv2, _s2u = _stats(a2_, _K1R, m2)
    pb = params["bottom"]
    hb = _gcn(a2_, h2, dinv2, pb["W"], pb["b"], scale2=sc1, m_col=m2,
              cast=True)

    pu0 = params["up0"]
    hs0m = _gcn(a1, hb, dinv1, pu0["W"], pu0["b"], m_col=m1, skip=h2,
                cast=True)

    pu1 = params["up1"]
    hs1, hs2 = _gcn(a0, hs0m, dinv0, pu1["W"], pu1["b"], skip=h1, org=h,
                    cast=True)

    hs0 = _gather_sc(hs0m, idx0)[:_K0R]

    return (hs0, hs1, hs2)


# speedup vs baseline: 2.4201x; 1.0984x over previous
from functools import partial

import jax
import jax.numpy as jnp
from jax import lax
from jax.experimental import pallas as pl
from jax.experimental.pallas import tpu as pltpu
from jax.experimental.pallas import tpu_sc as plsc

f32 = jnp.float32
bf16 = jnp.bfloat16
i32 = jnp.int32

_N = 2048
_DIM = 512
_HID = 128
_HEADS = 2
_HD = 64
_K0R, _K0P = 1638, 1664
_K1R = 982

_BM = 128
_NB = _N // _BM


def _statcols(s, dinv_ref, s2_ref, nm1, m=None):
    dinv = lax.rsqrt(1.0 + s)
    if m is not None:
        dinv = dinv * m
    s2 = jax.nn.sigmoid(3.0 * s / nm1)
    dinv_ref[...] = jnp.broadcast_to(dinv, dinv_ref.shape)
    s2_ref[...] = jnp.broadcast_to(s2, s2_ref.shape)


def _binarize_body(nm1, g_ref, a_ref, dinv_ref, s2_ref):
    a = (g_ref[...] != 0).astype(bf16)
    a_ref[...] = a
    s = jnp.sum(a.astype(f32), axis=1, keepdims=True)
    _statcols(s, dinv_ref, s2_ref, nm1)


def _binarize(g):
    return pl.pallas_call(
        partial(_binarize_body, float(_N - 1)),
        grid=(_NB,),
        in_specs=[pl.BlockSpec((_BM, _N), lambda i: (i, 0))],
        out_specs=[pl.BlockSpec((_BM, _N), lambda i: (i, 0)),
                   pl.BlockSpec((_BM, 128), lambda i: (i, 0)),
                   pl.BlockSpec((_BM, 128), lambda i: (i, 0))],
        out_shape=[jax.ShapeDtypeStruct((_N, _N), bf16),
                   jax.ShapeDtypeStruct((_N, 128), f32),
                   jax.ShapeDtypeStruct((_N, 128), f32)],
    )(g)


def _xw_body(has_s2, cast, *refs):
    if has_s2:
        x_ref, w_ref, s1_ref, s2_ref, o_ref = refs
    else:
        (x_ref, w_ref, s1_ref, o_ref), s2_ref = refs, None
    x, w = x_ref[...], w_ref[...]
    if cast:
        x, w = x.astype(bf16), w.astype(bf16)
    z = jnp.dot(x, w, preferred_element_type=f32)
    scale = s1_ref[...][:, :1]
    if s2_ref is not None:
        scale = scale * s2_ref[...][:, :1]
    o_ref[...] = z * scale


def _xw(x, w, scale1, scale2=None, cast=False):
    d_in, d_out = w.shape
    specs = [pl.BlockSpec((_BM, d_in), lambda i: (i, 0)),
             pl.BlockSpec((d_in, d_out), lambda i: (0, 0)),
             pl.BlockSpec((_BM, 128), lambda i: (i, 0))]
    args = [x, w, scale1]
    if scale2 is not None:
        specs.append(pl.BlockSpec((_BM, 128), lambda i: (i, 0)))
        args.append(scale2)
    return pl.pallas_call(
        partial(_xw_body, scale2 is not None, cast), grid=(_NB,),
        in_specs=specs,
        out_specs=pl.BlockSpec((_BM, d_out), lambda i: (i, 0)),
        out_shape=jax.ShapeDtypeStruct((_N, d_out), f32),
    )(*args)


def _adj_body(has_m, has_skip, has_org, cast, *refs):
    refs = list(refs)
    a_ref, z_ref, zd_ref, dinv_ref, b_ref = refs[:5]
    pos = 5
    m_ref = refs[pos] if has_m else None
    pos += int(has_m)
    skip_ref = refs[pos] if has_skip else None
    pos += int(has_skip)
    org_ref = refs[pos] if has_org else None
    pos += int(has_org)
    o_ref = refs[pos]
    o2_ref = refs[pos + 1] if has_org else None
    if cast:
        acc = jnp.dot(a_ref[...].astype(bf16), z_ref[...].astype(bf16),
                      preferred_element_type=f32)
    else:
        acc = jnp.dot(a_ref[...].astype(f32), z_ref[...],
                      preferred_element_type=f32)
    acc = acc + zd_ref[...]
    out = jax.nn.relu(acc * dinv_ref[...][:, :1] + b_ref[...])
    if m_ref is not None:
        out = out * m_ref[...][:, :1]
    if skip_ref is not None:
        out = out + skip_ref[...]
    o_ref[...] = out
    if o2_ref is not None:
        o2_ref[...] = out + org_ref[...]


def _adj(a, z, dinv, b, m_col=None, skip=None, org=None, cast=False):
    d = z.shape[1]
    specs = [pl.BlockSpec((_BM, _N), lambda i: (i, 0)),
             pl.BlockSpec((_N, d), lambda i: (0, 0)),
             pl.BlockSpec((_BM, d), lambda i: (i, 0)),
             pl.BlockSpec((_BM, 128), lambda i: (i, 0)),
             pl.BlockSpec((1, d), lambda i: (0, 0))]
    args = [a, z, z, dinv, b.reshape(1, d)]
    for extra in (m_col, skip, org):
        if extra is not None:
            specs.append(pl.BlockSpec((_BM, extra.shape[1]), lambda i: (i, 0)))
            args.append(extra)
    out_specs = [pl.BlockSpec((_BM, d), lambda i: (i, 0))]
    out_shape = [jax.ShapeDtypeStruct((_N, d), f32)]
    if org is not None:
        out_specs.append(pl.BlockSpec((_BM, d), lambda i: (i, 0)))
        out_shape.append(jax.ShapeDtypeStruct((_N, d), f32))
    body = partial(_adj_body, m_col is not None, skip is not None,
                   org is not None, cast)
    outs = pl.pallas_call(
        body, grid=(_NB,),
        in_specs=specs, out_specs=out_specs, out_shape=out_shape,
    )(*args)
    return outs if org is not None else outs[0]


def _qkv_body(x_ref, wq_ref, wk_ref, wv_ref, bq_ref, bk_ref, bv_ref,
              q_ref, k_ref, v_ref):
    x = x_ref[...].astype(bf16)
    q_ref[...] = jnp.dot(x, wq_ref[...].astype(bf16),
                         preferred_element_type=f32) + bq_ref[...]
    k_ref[...] = jnp.dot(x, wk_ref[...].astype(bf16),
                         preferred_element_type=f32) + bk_ref[...]
    v_ref[...] = jnp.dot(x, wv_ref[...].astype(bf16),
                         preferred_element_type=f32) + bv_ref[...]


def _qkv(x, p):
    wspec = pl.BlockSpec((_DIM, _HID), lambda i: (0, 0))
    bspec = pl.BlockSpec((1, _HID), lambda i: (0, 0))
    ospec = pl.BlockSpec((_BM, _HID), lambda i: (i, 0))
    return pl.pallas_call(
        _qkv_body, grid=(_NB,),
        in_specs=[pl.BlockSpec((_BM, _DIM), lambda i: (i, 0)),
                  wspec, wspec, wspec, bspec, bspec, bspec],
        out_specs=[ospec, ospec, ospec],
        out_shape=[jax.ShapeDtypeStruct((_N, _HID), f32)] * 3,
    )(x, p["Wq"], p["Wk"], p["Wv"], p["bq"].reshape(1, _HID),
      p["bk"].reshape(1, _HID), p["bv"].reshape(1, _HID))


def _attn_body(has_m, *refs):
    if has_m:
        q_ref, k_ref, v_ref, m_ref, o_ref = refs
    else:
        (q_ref, k_ref, v_ref, o_ref), m_ref = refs, None
    for hh in range(_HEADS):
        sl = slice(hh * _HD, (hh + 1) * _HD)
        s = lax.dot_general(q_ref[:, sl].astype(bf16),
                            k_ref[:, sl].astype(bf16),
                            (((1,), (1,)), ((), ())),
                            preferred_element_type=f32) * (1.0 / 8.0)
        if m_ref is not None:
            s = jnp.where(m_ref[...] > 0, s, -1e30)
        mx = jnp.max(s, axis=1, keepdims=True)
        p = jnp.exp(s - mx)
        l = jnp.sum(p, axis=1, keepdims=True)
        o_ref[:, sl] = jnp.dot(p.astype(bf16), v_ref[:, sl].astype(bf16),
                               preferred_element_type=f32) / l


def _attn(q, k, v, m_row=None):
    full = pl.BlockSpec((_N, _HID), lambda i: (0, 0))
    specs = [pl.BlockSpec((_BM, _HID), lambda i: (i, 0)), full, full]
    args = [q, k, v]
    if m_row is not None:
        specs.append(pl.BlockSpec((1, _N), lambda i: (0, 0)))
        args.append(m_row)
    return pl.pallas_call(
        partial(_attn_body, m_row is not None), grid=(_NB,),
        in_specs=specs,
        out_specs=pl.BlockSpec((_BM, _HID), lambda i: (i, 0)),
        out_shape=jax.ShapeDtypeStruct((_N, _HID), f32),
    )(*args)


def _combine_body(has_m, *refs):
    if has_m:
        ctx_ref, wd_ref, s2_ref, m_ref, bd_ref, va_ref, vb_ref, sc_ref = refs
    else:
        (ctx_ref, wd_ref, s2_ref, bd_ref, va_ref, vb_ref, sc_ref), m_ref = \
            refs, None
    raw = jnp.sum(ctx_ref[...] * wd_ref[...], axis=1, keepdims=True) \
        + bd_ref[0, 0]
    s1 = jax.nn.sigmoid(raw)
    s2 = s2_ref[...][:, :1]
    if m_ref is not None:
        valid = m_ref[...][:, :1] > 0
        s1 = jnp.where(valid, s1, 0.0)
        s2 = jnp.where(valid, s2, 0.0)
    sn1 = s1 / jnp.max(s1)
    sn2 = s2 / jnp.max(s2)
    a0 = jax.nn.sigmoid(sn1 * va_ref[0, 0] + sn2 * va_ref[1, 0] + vb_ref[0, 0])
    a1 = jax.nn.sigmoid(sn1 * va_ref[0, 1] + sn2 * va_ref[1, 1] + vb_ref[0, 1])
    mx = jnp.maximum(a0, a1)
    e0 = jnp.exp(a0 - mx)
    e1 = jnp.exp(a1 - mx)
    sc = jax.nn.sigmoid((sn1 * e0 + sn2 * e1) / (e0 + e1))
    if m_ref is not None:
        sc = jnp.where(valid, sc, -1e30)
    sc_ref[...] = jnp.broadcast_to(sc, sc_ref.shape)


def _combine(ctx, s2_col, p, m_col=None):
    specs = [pl.BlockSpec((_N, _HID), lambda: (0, 0)),
             pl.BlockSpec((1, _HID), lambda: (0, 0)),
             pl.BlockSpec((_N, 128), lambda: (0, 0))]
    args = [ctx, p["Wd"].reshape(1, _HID), s2_col]
    if m_col is not None:
        specs.append(pl.BlockSpec((_N, 128), lambda: (0, 0)))
        args.append(m_col)
    specs += [pl.BlockSpec(memory_space=pltpu.SMEM)] * 3
    args += [p["bd"].reshape(1, 1), p["view_att"], p["view_bias"].reshape(1, 2)]
    return pl.pallas_call(
        partial(_combine_body, m_col is not None),
        in_specs=specs,
        out_specs=pl.BlockSpec((_N, 128), lambda: (0, 0)),
        out_shape=jax.ShapeDtypeStruct((_N, 128), f32),
    )(*args)


def _rank_body(kk_real, sc_col_ref, sc_row_ref, r_ref, m_ref):
    i = pl.program_id(0)
    s_i = sc_col_ref[...][:, :1]
    s_j = sc_row_ref[...]
    jj = lax.broadcasted_iota(i32, s_j.shape, 1)
    ii = i * _BM + lax.broadcasted_iota(i32, (_BM, 1), 0)
    beats = (s_j > s_i) | ((s_j == s_i) & (jj < ii))
    r = jnp.sum(beats.astype(i32), axis=1, keepdims=True)
    r_ref[...] = jnp.broadcast_to(r, r_ref.shape)
    m = (r < kk_real).astype(f32)
    m_ref[...] = jnp.broadcast_to(m, m_ref.shape)


def _rank(sc_col, sc_row, kk_real):
    return pl.pallas_call(
        partial(_rank_body, kk_real), grid=(_NB,),
        in_specs=[pl.BlockSpec((_BM, 128), lambda i: (i, 0)),
                  pl.BlockSpec((1, _N), lambda i: (0, 0))],
        out_specs=[pl.BlockSpec((_BM, 128), lambda i: (i, 0)),
                   pl.BlockSpec((_BM, 128), lambda i: (i, 0))],
        out_shape=[jax.ShapeDtypeStruct((_N, 128), i32),
                   jax.ShapeDtypeStruct((_N, 128), f32)],
    )(sc_col, sc_row)


def _idxsel_body(rank_row_ref, idx_ref):
    i = pl.program_id(0)
    rr = rank_row_ref[...]
    r_col = i * _BM + lax.broadcasted_iota(i32, (_BM, 1), 0)
    ii = lax.broadcasted_iota(i32, rr.shape, 1)
    hit = jnp.where(rr == r_col, ii, 0)
    idx = jnp.sum(hit, axis=1, keepdims=True)
    idx_ref[...] = jnp.broadcast_to(idx, idx_ref.shape)


def _idxsel(rank_row, kk_pad):
    return pl.pallas_call(
        _idxsel_body, grid=(kk_pad // _BM,),
        in_specs=[pl.BlockSpec((1, _N), lambda i: (0, 0))],
        out_specs=pl.BlockSpec((_BM, 128), lambda i: (i, 0)),
        out_shape=jax.ShapeDtypeStruct((kk_pad, 128), i32),
    )(rank_row)


def _a2_body(nm1, a_ref, ac_ref, m_ref, mr_ref, o_ref, dinv_ref, s2_ref):
    acc = jnp.dot(a_ref[...], ac_ref[...], preferred_element_type=f32)
    m_i = m_ref[...][:, :1]
    keep = (acc > 0.5) & (m_i > 0) & (mr_ref[...] > 0)
    keep_f = keep.astype(f32)
    o_ref[...] = keep_f.astype(bf16)
    s = jnp.sum(keep_f, axis=1, keepdims=True)
    _statcols(s, dinv_ref, s2_ref, nm1, m=m_i)


def _a2(a, m_col, m_row, n_real):
    return pl.pallas_call(
        partial(_a2_body, float(n_real - 1)), grid=(_NB,),
        in_specs=[pl.BlockSpec((_BM, _N), lambda i: (i, 0)),
                  pl.BlockSpec((_N, _N), lambda i: (0, 0)),
                  pl.BlockSpec((_BM, 128), lambda i: (i, 0)),
                  pl.BlockSpec((1, _N), lambda i: (0, 0))],
        out_specs=[pl.BlockSpec((_BM, _N), lambda i: (i, 0)),
                   pl.BlockSpec((_BM, 128), lambda i: (i, 0)),
                   pl.BlockSpec((_BM, 128), lambda i: (i, 0))],
        out_shape=[jax.ShapeDtypeStruct((_N, _N), bf16),
                   jax.ShapeDtypeStruct((_N, 128), f32),
                   jax.ShapeDtypeStruct((_N, 128), f32)],
    )(a, a, m_col, m_row)


def _sc_mesh():
    return plsc.VectorSubcoreMesh(core_axis_name="c", subcore_axis_name="s",
                                  num_cores=2, num_subcores=16)


def _gather_sc(table, idx, rows_per_tile=64):
    out_rows = idx.shape[0]
    n_tiles = out_rows // rows_per_tile
    row_w = table.shape[1]
    dtype = table.dtype
    idx2d = idx.reshape(n_tiles, rows_per_tile)

    @partial(pl.kernel,
             out_type=jax.ShapeDtypeStruct((out_rows, row_w), dtype),
             mesh=_sc_mesh(),
             scratch_types=[pltpu.VMEM((rows_per_tile,), i32),
                            pltpu.VMEM((rows_per_tile, row_w), dtype),
                            pltpu.SemaphoreType.DMA])
    def k(tab_hbm, idx_hbm, out_hbm, idx_v, rows_v, sem):
        wid = lax.axis_index("s") * 2 + lax.axis_index("c")

        @pl.when(wid < n_tiles)
        def _():
            pltpu.sync_copy(idx_hbm.at[wid], idx_v)
            pltpu.async_copy(tab_hbm.at[idx_v], rows_v, sem).wait()
            pltpu.sync_copy(
                rows_v, out_hbm.at[pl.ds(wid * rows_per_tile, rows_per_tile)])

    return k(table, idx2d)


def _gcn(a, x, dinv, w, b, scale2=None, m_col=None, skip=None, org=None,
         cast=False):
    z = _xw(x, w, dinv, scale2, cast=cast)
    return _adj(a, z, dinv, b, m_col=m_col, skip=skip, org=org, cast=cast)


def _pool_scores(hh, p, s2_col, m_col=None, m_row=None):
    q, k, v = _qkv(hh, p)
    ctx = _attn(q, k, v, m_row)
    return _combine(ctx, s2_col, p, m_col)


def _row(col):
    return col[:, 0][None, :]


def kernel(g, h, params):
    g = jnp.asarray(g, f32)
    h = jnp.asarray(h, f32)

    a0, dinv0, s2c0 = _binarize(g)
    p0 = params["down0"]
    h1 = _gcn(a0, h, dinv0, p0["W"], p0["b"], cast=True)
    sc0 = _pool_scores(h1, params["pool0"], s2c0)
    r0, m1 = _rank(sc0, _row(sc0), _K0R)
    idx0 = _idxsel(_row(r0), _K0P)[:, 0]
    m1row = _row(m1)

    a1, dinv1, s2c1 = _a2(a0, m1, m1row, _K0R)

    p1 = params["down1"]
    h2 = _gcn(a1, h1, dinv1, p1["W"], p1["b"], scale2=sc0, m_col=m1,
              cast=True)
    sc1 = _pool_scores(h2, params["pool1"], s2c1, m_col=m1, m_row=m1row)
    _r1, m2 = _rank(sc1, _row(sc1), _K1R)

    a2_, dinv2, _s2u = _a2(a1, m2, _row(m2), _K1R)

    pb = params["bottom"]
    hb = _gcn(a2_, h2, dinv2, pb["W"], pb["b"], scale2=sc1, m_col=m2,
              cast=True)

    pu0 = params["up0"]
    hs0m = _gcn(a1, hb, dinv1, pu0["W"], pu0["b"], m_col=m1, skip=h2,
                cast=True)

    pu1 = params["up1"]
    hs1, hs2 = _gcn(a0, hs0m, dinv0, pu1["W"], pu1["b"], skip=h1, org=h,
                    cast=True)

    hs0 = _gather_sc(hs0m, idx0)[:_K0R]

    return (hs0, hs1, hs2)


# speedup vs baseline: 3.0246x; 1.2498x over previous
from functools import partial

import jax
import jax.numpy as jnp
from jax import lax
from jax.experimental import pallas as pl
from jax.experimental.pallas import tpu as pltpu
from jax.experimental.pallas import tpu_sc as plsc

f32 = jnp.float32
bf16 = jnp.bfloat16
i32 = jnp.int32

_N = 2048
_DIM = 512
_HID = 128
_HEADS = 2
_HD = 64
_K0R, _K0P = 1638, 1664
_K1R = 982

_BM = 256
_NB = _N // _BM


def _statcols(s, dinv_ref, s2_ref, nm1, m=None):
    dinv = lax.rsqrt(1.0 + s)
    if m is not None:
        dinv = dinv * m
    s2 = jax.nn.sigmoid(3.0 * s / nm1)
    dinv_ref[...] = jnp.broadcast_to(dinv, dinv_ref.shape)
    s2_ref[...] = jnp.broadcast_to(s2, s2_ref.shape)


def _binarize_body(nm1, g_ref, a_ref, dinv_ref, s2_ref):
    a = (g_ref[...] != 0).astype(bf16)
    a_ref[...] = a
    s = jnp.sum(a.astype(f32), axis=1, keepdims=True)
    _statcols(s, dinv_ref, s2_ref, nm1)


def _binarize(g):
    return pl.pallas_call(
        partial(_binarize_body, float(_N - 1)),
        grid=(_NB,),
        in_specs=[pl.BlockSpec((_BM, _N), lambda i: (i, 0))],
        out_specs=[pl.BlockSpec((_BM, _N), lambda i: (i, 0)),
                   pl.BlockSpec((_BM, 128), lambda i: (i, 0)),
                   pl.BlockSpec((_BM, 128), lambda i: (i, 0))],
        out_shape=[jax.ShapeDtypeStruct((_N, _N), bf16),
                   jax.ShapeDtypeStruct((_N, 128), f32),
                   jax.ShapeDtypeStruct((_N, 128), f32)],
    )(g)


def _xw_body(has_s2, cast, *refs):
    if has_s2:
        x_ref, w_ref, s1_ref, s2_ref, o_ref = refs
    else:
        (x_ref, w_ref, s1_ref, o_ref), s2_ref = refs, None
    x, w = x_ref[...], w_ref[...]
    if cast:
        x, w = x.astype(bf16), w.astype(bf16)
    z = jnp.dot(x, w, preferred_element_type=f32)
    scale = s1_ref[...][:, :1]
    if s2_ref is not None:
        scale = scale * s2_ref[...][:, :1]
    o_ref[...] = z * scale


def _xw(x, w, scale1, scale2=None, cast=False):
    d_in, d_out = w.shape
    specs = [pl.BlockSpec((_BM, d_in), lambda i: (i, 0)),
             pl.BlockSpec((d_in, d_out), lambda i: (0, 0)),
             pl.BlockSpec((_BM, 128), lambda i: (i, 0))]
    args = [x, w, scale1]
    if scale2 is not None:
        specs.append(pl.BlockSpec((_BM, 128), lambda i: (i, 0)))
        args.append(scale2)
    return pl.pallas_call(
        partial(_xw_body, scale2 is not None, cast), grid=(_NB,),
        in_specs=specs,
        out_specs=pl.BlockSpec((_BM, d_out), lambda i: (i, 0)),
        out_shape=jax.ShapeDtypeStruct((_N, d_out), f32),
    )(*args)


def _adj_body(has_m, has_skip, has_org, cast, *refs):
    refs = list(refs)
    a_ref, z_ref, zd_ref, dinv_ref, b_ref = refs[:5]
    pos = 5
    m_ref = refs[pos] if has_m else None
    pos += int(has_m)
    skip_ref = refs[pos] if has_skip else None
    pos += int(has_skip)
    org_ref = refs[pos] if has_org else None
    pos += int(has_org)
    o_ref = refs[pos]
    o2_ref = refs[pos + 1] if has_org else None
    if cast:
        acc = jnp.dot(a_ref[...].astype(bf16), z_ref[...].astype(bf16),
                      preferred_element_type=f32)
    else:
        acc = jnp.dot(a_ref[...].astype(f32), z_ref[...],
                      preferred_element_type=f32)
    acc = acc + zd_ref[...]
    out = jax.nn.relu(acc * dinv_ref[...][:, :1] + b_ref[...])
    if m_ref is not None:
        out = out * m_ref[...][:, :1]
    if skip_ref is not None:
        out = out + skip_ref[...]
    o_ref[...] = out
    if o2_ref is not None:
        o2_ref[...] = out + org_ref[...]


def _adj(a, z, dinv, b, m_col=None, skip=None, org=None, cast=False):
    d = z.shape[1]
    specs = [pl.BlockSpec((_BM, _N), lambda i: (i, 0)),
             pl.BlockSpec((_N, d), lambda i: (0, 0)),
             pl.BlockSpec((_BM, d), lambda i: (i, 0)),
             pl.BlockSpec((_BM, 128), lambda i: (i, 0)),
             pl.BlockSpec((1, d), lambda i: (0, 0))]
    args = [a, z, z, dinv, b.reshape(1, d)]
    for extra in (m_col, skip, org):
        if extra is not None:
            specs.append(pl.BlockSpec((_BM, extra.shape[1]), lambda i: (i, 0)))
            args.append(extra)
    out_specs = [pl.BlockSpec((_BM, d), lambda i: (i, 0))]
    out_shape = [jax.ShapeDtypeStruct((_N, d), f32)]
    if org is not None:
        out_specs.append(pl.BlockSpec((_BM, d), lambda i: (i, 0)))
        out_shape.append(jax.ShapeDtypeStruct((_N, d), f32))
    body = partial(_adj_body, m_col is not None, skip is not None,
                   org is not None, cast)
    outs = pl.pallas_call(
        body, grid=(_NB,),
        in_specs=specs, out_specs=out_specs, out_shape=out_shape,
    )(*args)
    return outs if org is not None else outs[0]


def _qkv_body(x_ref, w_ref, b_ref, q_ref, k_ref, v_ref):
    x = x_ref[...].astype(bf16)
    qkv = jnp.dot(x, w_ref[...].astype(bf16),
                  preferred_element_type=f32) + b_ref[...]
    q_ref[...] = qkv[:, :_HID]
    k_ref[...] = qkv[:, _HID:2 * _HID]
    v_ref[...] = qkv[:, 2 * _HID:]


def _qkv(x, p):
    w = jnp.concatenate([p["Wq"], p["Wk"], p["Wv"]], axis=1)
    b = jnp.concatenate([p["bq"], p["bk"], p["bv"]]).reshape(1, 3 * _HID)
    ospec = pl.BlockSpec((_BM, _HID), lambda i: (i, 0))
    return pl.pallas_call(
        _qkv_body, grid=(_NB,),
        in_specs=[pl.BlockSpec((_BM, _DIM), lambda i: (i, 0)),
                  pl.BlockSpec((_DIM, 3 * _HID), lambda i: (0, 0)),
                  pl.BlockSpec((1, 3 * _HID), lambda i: (0, 0))],
        out_specs=[ospec, ospec, ospec],
        out_shape=[jax.ShapeDtypeStruct((_N, _HID), f32)] * 3,
    )(x, w, b)


def _attn_body(has_m, *refs):
    if has_m:
        q_ref, k_ref, v_ref, wd_ref, m_ref, o_ref = refs
    else:
        (q_ref, k_ref, v_ref, wd_ref, o_ref), m_ref = refs, None
    parts = []
    for hh in range(_HEADS):
        sl = slice(hh * _HD, (hh + 1) * _HD)
        s = lax.dot_general(q_ref[:, sl].astype(bf16),
                            k_ref[:, sl].astype(bf16),
                            (((1,), (1,)), ((), ())),
                            preferred_element_type=f32) * (1.0 / 8.0)
        if m_ref is not None:
            s = jnp.where(m_ref[...] > 0, s, -1e30)
        mx = jnp.max(s, axis=1, keepdims=True)
        p = jnp.exp(s - mx)
        l = jnp.sum(p, axis=1, keepdims=True)
        ctx = jnp.dot(p.astype(bf16), v_ref[:, sl].astype(bf16),
                      preferred_element_type=f32) / l
        parts.append(jnp.sum(ctx * wd_ref[:, sl], axis=1, keepdims=True))
    raw = parts[0] + parts[1]
    o_ref[...] = jnp.broadcast_to(raw, o_ref.shape)


def _attn(q, k, v, wd_row, m_row=None):
    full = pl.BlockSpec((_N, _HID), lambda i: (0, 0))
    specs = [pl.BlockSpec((_BM, _HID), lambda i: (i, 0)), full, full,
             pl.BlockSpec((1, _HID), lambda i: (0, 0))]
    args = [q, k, v, wd_row]
    if m_row is not None:
        specs.append(pl.BlockSpec((1, _N), lambda i: (0, 0)))
        args.append(m_row)
    return pl.pallas_call(
        partial(_attn_body, m_row is not None), grid=(_NB,),
        in_specs=specs,
        out_specs=pl.BlockSpec((_BM, 128), lambda i: (i, 0)),
        out_shape=jax.ShapeDtypeStruct((_N, 128), f32),
    )(*args)


def _combine_body(has_m, *refs):
    if has_m:
        raw_ref, s2_ref, m_ref, bd_ref, va_ref, vb_ref, sc_ref = refs
    else:
        (raw_ref, s2_ref, bd_ref, va_ref, vb_ref, sc_ref), m_ref = \
            refs, None
    raw = raw_ref[...][:, :1] + bd_ref[0, 0]
    s1 = jax.nn.sigmoid(raw)
    s2 = s2_ref[...][:, :1]
    if m_ref is not None:
        valid = m_ref[...][:, :1] > 0
        s1 = jnp.where(valid, s1, 0.0)
        s2 = jnp.where(valid, s2, 0.0)
    sn1 = s1 / jnp.max(s1)
    sn2 = s2 / jnp.max(s2)
    a0 = jax.nn.sigmoid(sn1 * va_ref[0, 0] + sn2 * va_ref[1, 0] + vb_ref[0, 0])
    a1 = jax.nn.sigmoid(sn1 * va_ref[0, 1] + sn2 * va_ref[1, 1] + vb_ref[0, 1])
    mx = jnp.maximum(a0, a1)
    e0 = jnp.exp(a0 - mx)
    e1 = jnp.exp(a1 - mx)
    sc = jax.nn.sigmoid((sn1 * e0 + sn2 * e1) / (e0 + e1))
    if m_ref is not None:
        sc = jnp.where(valid, sc, -1e30)
    sc_ref[...] = jnp.broadcast_to(sc, sc_ref.shape)


def _combine(raw_col, s2_col, p, m_col=None):
    specs = [pl.BlockSpec((_N, 128), lambda: (0, 0)),
             pl.BlockSpec((_N, 128), lambda: (0, 0))]
    args = [raw_col, s2_col]
    if m_col is not None:
        specs.append(pl.BlockSpec((_N, 128), lambda: (0, 0)))
        args.append(m_col)
    specs += [pl.BlockSpec(memory_space=pltpu.SMEM)] * 3
    args += [p["bd"].reshape(1, 1), p["view_att"], p["view_bias"].reshape(1, 2)]
    return pl.pallas_call(
        partial(_combine_body, m_col is not None),
        in_specs=specs,
        out_specs=pl.BlockSpec((_N, 128), lambda: (0, 0)),
        out_shape=jax.ShapeDtypeStruct((_N, 128), f32),
    )(*args)


def _rank_body(kk_real, sc_col_ref, sc_row_ref, r_ref, m_ref):
    i = pl.program_id(0)
    s_i = sc_col_ref[...][:, :1]
    s_j = sc_row_ref[...]
    jj = lax.broadcasted_iota(i32, s_j.shape, 1)
    ii = i * _BM + lax.broadcasted_iota(i32, (_BM, 1), 0)
    beats = (s_j > s_i) | ((s_j == s_i) & (jj < ii))
    r = jnp.sum(beats.astype(i32), axis=1, keepdims=True)
    r_ref[...] = jnp.broadcast_to(r, r_ref.shape)
    m = (r < kk_real).astype(f32)
    m_ref[...] = jnp.broadcast_to(m, m_ref.shape)


def _rank(sc_col, sc_row, kk_real):
    return pl.pallas_call(
        partial(_rank_body, kk_real), grid=(_NB,),
        in_specs=[pl.BlockSpec((_BM, 128), lambda i: (i, 0)),
                  pl.BlockSpec((1, _N), lambda i: (0, 0))],
        out_specs=[pl.BlockSpec((_BM, 128), lambda i: (i, 0)),
                   pl.BlockSpec((_BM, 128), lambda i: (i, 0))],
        out_shape=[jax.ShapeDtypeStruct((_N, 128), i32),
                   jax.ShapeDtypeStruct((_N, 128), f32)],
    )(sc_col, sc_row)


def _idxsel_body(rank_row_ref, idx_ref):
    i = pl.program_id(0)
    rr = rank_row_ref[...]
    r_col = i * _BM + lax.broadcasted_iota(i32, (_BM, 1), 0)
    ii = lax.broadcasted_iota(i32, rr.shape, 1)
    hit = jnp.where(rr == r_col, ii, 0)
    idx = jnp.sum(hit, axis=1, keepdims=True)
    idx_ref[...] = jnp.broadcast_to(idx, idx_ref.shape)


def _idxsel(rank_row, kk_pad):
    return pl.pallas_call(
        _idxsel_body, grid=(kk_pad // _BM,),
        in_specs=[pl.BlockSpec((1, _N), lambda i: (0, 0))],
        out_specs=pl.BlockSpec((_BM, 128), lambda i: (i, 0)),
        out_shape=jax.ShapeDtypeStruct((kk_pad, 128), i32),
    )(rank_row)


def _a2_body(nm1, a_ref, ac_ref, m_ref, mr_ref, o_ref, dinv_ref, s2_ref):
    acc = jnp.dot(a_ref[...], ac_ref[...], preferred_element_type=f32)
    m_i = m_ref[...][:, :1]
    keep = (acc > 0.5) & (m_i > 0) & (mr_ref[...] > 0)
    keep_f = keep.astype(f32)
    o_ref[...] = keep_f.astype(bf16)
    s = jnp.sum(keep_f, axis=1, keepdims=True)
    _statcols(s, dinv_ref, s2_ref, nm1, m=m_i)


def _a2(a, m_col, m_row, n_real):
    return pl.pallas_call(
        partial(_a2_body, float(n_real - 1)), grid=(_NB,),
        in_specs=[pl.BlockSpec((_BM, _N), lambda i: (i, 0)),
                  pl.BlockSpec((_N, _N), lambda i: (0, 0)),
                  pl.BlockSpec((_BM, 128), lambda i: (i, 0)),
                  pl.BlockSpec((1, _N), lambda i: (0, 0))],
        out_specs=[pl.BlockSpec((_BM, _N), lambda i: (i, 0)),
                   pl.BlockSpec((_BM, 128), lambda i: (i, 0)),
                   pl.BlockSpec((_BM, 128), lambda i: (i, 0))],
        out_shape=[jax.ShapeDtypeStruct((_N, _N), bf16),
                   jax.ShapeDtypeStruct((_N, 128), f32),
                   jax.ShapeDtypeStruct((_N, 128), f32)],
    )(a, a, m_col, m_row)


def _sc_mesh():
    return plsc.VectorSubcoreMesh(core_axis_name="c", subcore_axis_name="s",
                                  num_cores=2, num_subcores=16)


def _gather_sc(table, idx, rows_per_tile=64):
    out_rows = idx.shape[0]
    n_tiles = out_rows // rows_per_tile
    row_w = table.shape[1]
    dtype = table.dtype
    idx2d = idx.reshape(n_tiles, rows_per_tile)

    @partial(pl.kernel,
             out_type=jax.ShapeDtypeStruct((out_rows, row_w), dtype),
             mesh=_sc_mesh(),
             scratch_types=[pltpu.VMEM((rows_per_tile,), i32),
                            pltpu.VMEM((rows_per_tile, row_w), dtype),
                            pltpu.SemaphoreType.DMA])
    def k(tab_hbm, idx_hbm, out_hbm, idx_v, rows_v, sem):
        wid = lax.axis_index("s") * 2 + lax.axis_index("c")

        @pl.when(wid < n_tiles)
        def _():
            pltpu.sync_copy(idx_hbm.at[wid], idx_v)
            pltpu.async_copy(tab_hbm.at[idx_v], rows_v, sem).wait()
            pltpu.sync_copy(
                rows_v, out_hbm.at[pl.ds(wid * rows_per_tile, rows_per_tile)])

    return k(table, idx2d)


def _gcn(a, x, dinv, w, b, scale2=None, m_col=None, skip=None, org=None,
         cast=False):
    z = _xw(x, w, dinv, scale2, cast=cast)
    return _adj(a, z, dinv, b, m_col=m_col, skip=skip, org=org, cast=cast)


def _pool_scores(hh, p, s2_col, m_col=None, m_row=None):
    q, k, v = _qkv(hh, p)
    raw = _attn(q, k, v, p["Wd"].reshape(1, _HID), m_row)
    return _combine(raw, s2_col, p, m_col)


def _row(col):
    return col[:, 0][None, :]


def kernel(g, h, params):
    g = jnp.asarray(g, f32)
    h = jnp.asarray(h, f32)

    a0, dinv0, s2c0 = _binarize(g)
    p0 = params["down0"]
    h1 = _gcn(a0, h, dinv0, p0["W"], p0["b"], cast=True)
    sc0 = _pool_scores(h1, params["pool0"], s2c0)
    r0, m1 = _rank(sc0, _row(sc0), _K0R)
    idx0 = _idxsel(_row(r0), _K0P)[:, 0]
    m1row = _row(m1)

    a1, dinv1, s2c1 = _a2(a0, m1, m1row, _K0R)

    p1 = params["down1"]
    h2 = _gcn(a1, h1, dinv1, p1["W"], p1["b"], scale2=sc0, m_col=m1,
              cast=True)
    sc1 = _pool_scores(h2, params["pool1"], s2c1, m_col=m1, m_row=m1row)
    _r1, m2 = _rank(sc1, _row(sc1), _K1R)

    a2_, dinv2, _s2u = _a2(a1, m2, _row(m2), _K1R)

    pb = params["bottom"]
    hb = _gcn(a2_, h2, dinv2, pb["W"], pb["b"], scale2=sc1, m_col=m2,
              cast=True)

    pu0 = params["up0"]
    hs0m = _gcn(a1, hb, dinv1, pu0["W"], pu0["b"], m_col=m1, skip=h2,
                cast=True)

    pu1 = params["up1"]
    hs1, hs2 = _gcn(a0, hs0m, dinv0, pu1["W"], pu1["b"], skip=h1, org=h,
                    cast=True)

    hs0 = _gather_sc(hs0m, idx0)[:_K0R]

    return (hs0, hs1, hs2)
